# Initial kernel scaffold; baseline (speedup 1.0000x reference)
import jax
import jax.numpy as jnp
from jax.experimental import pallas as pl


def kernel(text, offsets, emb_table, fc_w, fc_b):
    raise NotImplementedError("write your pallas kernel here")


# speedup vs baseline: 255.5420x; 255.5420x over previous
import functools

import jax
import jax.numpy as jnp
from jax import lax
from jax.experimental import pallas as pl
from jax.experimental.pallas import tpu as pltpu
from jax.experimental.pallas import tpu_sc as plsc

_NC = 2
_NS = 16
_L = 16
_NW = _NC * _NS

_CH = 16384


def _proj_body(emb_ref, w_ref, out_ref):
    out_ref[...] = jnp.dot(emb_ref[...], w_ref[...],
                           preferred_element_type=jnp.float32)


def _project(emb, wpad, vb):
    V, D = emb.shape
    CP = wpad.shape[1]
    return pl.pallas_call(
        _proj_body,
        grid=(V // vb,),
        in_specs=[
            pl.BlockSpec((vb, D), lambda i: (i, 0)),
            pl.BlockSpec((D, CP), lambda i: (0, 0)),
        ],
        out_specs=pl.BlockSpec((vb, CP), lambda i: (i, 0)),
        out_shape=jax.ShapeDtypeStruct((V, CP), jnp.float32),
    )(emb, wpad)


@functools.lru_cache(maxsize=None)
def _make_sc_kernel(T, B, V, C):
    bpw = B // _NW
    mesh = plsc.VectorSubcoreMesh(
        core_axis_name="c", subcore_axis_name="s",
        num_cores=_NC, num_subcores=_NS)

    @functools.partial(
        pl.kernel,
        out_type=jax.ShapeDtypeStruct((C * B,), jnp.float32),
        mesh=mesh,
        scratch_types=[
            pltpu.VMEM((V,), jnp.float32),
            pltpu.VMEM((_CH,), jnp.int32),
            pltpu.VMEM((bpw + 16,), jnp.int32),
            pltpu.VMEM((bpw,), jnp.float32),
            pltpu.VMEM((16,), jnp.float32),
        ],
        compiler_params=pltpu.CompilerParams(needs_layout_passes=False),
    )
    def sc_kernel(text_hbm, offs_hbm, projT_hbm, fcb_hbm, out_hbm,
                  tab_v, txt_v, off_v, line_v, fcb_v):
        wid = lax.axis_index("s") * _NC + lax.axis_index("c")
        b0 = wid * bpw
        pltpu.sync_copy(offs_hbm.at[pl.ds(b0, bpw + 16)], off_v)
        pltpu.sync_copy(fcb_hbm, fcb_v)
        lane = lax.iota(jnp.int32, _L)
        bias_vec = fcb_v[pl.ds(0, _L)]
        cur_chunk = jnp.int32(-1)

        for c in range(C):
            pltpu.sync_copy(projT_hbm.at[pl.ds(c * V, V)], tab_v)
            bias_c = bias_vec[c]

            def bag_body(b, cur, _bias=bias_c):
                offpair = off_v[pl.ds(b, _L)]
                lo = offpair[0]
                hi = offpair[1]
                i0 = lax.div(lo, 16)
                i1 = lax.div(hi + 15, 16)

                def blk_body(i, carry):
                    acc, cur = carry
                    ck = lax.div(i * 16, _CH)

                    @pl.when(ck != cur)
                    def _():
                        pltpu.sync_copy(
                            text_hbm.at[pl.ds(ck * _CH, _CH)], txt_v)

                    toks = txt_v[pl.ds(i * 16 - ck * _CH, _L)]
                    g = i * 16 + lane
                    m = (g >= lo) & (g < hi)
                    idx = jnp.where(m, toks, 0)
                    vals = plsc.load_gather(tab_v, [idx], mask=m)
                    acc = acc + jnp.where(m, vals, jnp.float32(0.0))
                    return (acc, ck)

                acc, cur = lax.fori_loop(
                    i0, i1, blk_body,
                    (jnp.zeros((_L,), jnp.float32), cur))
                plsc.store_scatter(
                    line_v, [jnp.full((_L,), b, jnp.int32)],
                    jnp.full((_L,), jnp.sum(acc), jnp.float32),
                    mask=lane == 0)
                return cur

            cur_chunk = lax.fori_loop(0, bpw, bag_body, cur_chunk)

            def mean_body(j, _, _bias=bias_c):
                sums = line_v[pl.ds(j * _L, _L)]
                o_lo = off_v[pl.ds(j * _L, _L)]
                o_hi = off_v[pl.ds(j * _L + 1, _L)]
                cnt = (o_hi - o_lo).astype(jnp.float32)
                line_v[pl.ds(j * _L, _L)] = (
                    sums / jnp.maximum(cnt, jnp.float32(1.0)) + _bias)
                return 0

            lax.fori_loop(0, bpw // _L, mean_body, 0)
            pltpu.sync_copy(line_v, out_hbm.at[pl.ds(c * B + b0, bpw)])

    return sc_kernel


def kernel(text, offsets, emb_table, fc_w, fc_b):
    T = text.shape[0]
    B = offsets.shape[0]
    V, D = emb_table.shape
    C = fc_w.shape[0]

    wpad = jnp.zeros((D, 8), jnp.float32).at[:, :C].set(fc_w.T)
    proj8 = _project(emb_table, wpad, 2000)
    projT = proj8[:, :C].T.reshape(-1)

    offs_ext = jnp.concatenate(
        [offsets.astype(jnp.int32), jnp.full((16,), T, jnp.int32)])
    fcb_pad = jnp.zeros((16,), jnp.float32).at[:C].set(fc_b)

    out_flat = _make_sc_kernel(T, B, V, C)(
        text.astype(jnp.int32), offs_ext, projT, fcb_pad)
    return out_flat.reshape(C, B).T


# speedup vs baseline: 292.5267x; 1.1447x over previous
import functools

import jax
import jax.numpy as jnp
from jax import lax
from jax.experimental import pallas as pl
from jax.experimental.pallas import tpu as pltpu
from jax.experimental.pallas import tpu_sc as plsc

_NC = 2
_NS = 16
_L = 16
_NW = _NC * _NS

_CP = 4
_CH = 8192


def _proj_body(emb_ref, w_ref, out_ref):
    out_ref[...] = jnp.dot(emb_ref[...], w_ref[...],
                           preferred_element_type=jnp.float32)


def _project(emb, wpad, vb):
    V, D = emb.shape
    return pl.pallas_call(
        _proj_body,
        grid=(V // vb,),
        in_specs=[
            pl.BlockSpec((vb, D), lambda i: (i, 0)),
            pl.BlockSpec((D, _CP), lambda i: (0, 0)),
        ],
        out_specs=pl.BlockSpec((vb, _CP), lambda i: (i, 0)),
        out_shape=jax.ShapeDtypeStruct((V, _CP), jnp.float32),
    )(emb, wpad)


@functools.lru_cache(maxsize=None)
def _make_sc_kernel(T, B, V, C):
    bpw = B // _NW

    mesh = plsc.VectorSubcoreMesh(
        core_axis_name="c", subcore_axis_name="s",
        num_cores=_NC, num_subcores=_NS)

    @functools.partial(
        pl.kernel,
        out_type=jax.ShapeDtypeStruct((C * B,), jnp.float32),
        mesh=mesh,
        scratch_types=[
            pltpu.VMEM((_CH,), jnp.int32),
            pltpu.VMEM((_CH, _CP), jnp.float32),
            pltpu.VMEM((bpw + 16,), jnp.int32),
            pltpu.VMEM((C * bpw,), jnp.float32),
            pltpu.VMEM((16,), jnp.float32),
            pltpu.SemaphoreType.DMA,
        ],
        compiler_params=pltpu.CompilerParams(
            needs_layout_passes=False, use_tc_tiling_on_sc=False),
    )
    def sc_kernel(text_hbm, offs_hbm, proj_hbm, fcb_hbm, out_hbm,
                  txt_v, rows_v, off_v, line_v, fcb_v, sem):
        wid = lax.axis_index("s") * _NC + lax.axis_index("c")
        b0 = wid * bpw
        pltpu.sync_copy(offs_hbm.at[pl.ds(b0, bpw + 16)], off_v)
        pltpu.sync_copy(fcb_hbm, fcb_v)
        lane = lax.iota(jnp.int32, _L)
        bias_vec = fcb_v[pl.ds(0, _L)]
        tq = lax.shift_right_logical(lane, 2)
        cq = lax.bitwise_and(lane, 3)
        perm8 = lax.bitwise_and(lane + 8, 15)
        perm4 = lax.bitwise_and(lane + 4, 15)
        store_idx_base = lane * bpw
        store_mask = lane < C

        def bag_body(b, cur):
            offpair = off_v[pl.ds(b, _L)]
            lo = offpair[0]
            hi = offpair[1]
            i0 = lax.div(lo, 16)
            i1 = lax.div(hi + 15, 16)

            def blk_body(i, carry):
                acc, cur = carry
                ck = lax.div(i * 16, _CH)

                @pl.when(ck != cur)
                def _():
                    pltpu.sync_copy(
                        text_hbm.at[pl.ds(ck * _CH, _CH)], txt_v)
                    pltpu.async_copy(
                        proj_hbm.at[txt_v], rows_v, sem).wait()

                local = i * 16 - ck * _CH
                for j in range(4):
                    tok = i * 16 + 4 * j + tq
                    m = (tok >= lo) & (tok < hi)
                    ridx = jnp.where(m, local + 4 * j + tq, 0)
                    vals = plsc.load_gather(rows_v, [ridx, cq], mask=m)
                    acc = acc + jnp.where(m, vals, jnp.float32(0.0))
                return (acc, ck)

            acc, cur = lax.fori_loop(
                i0, i1, blk_body,
                (jnp.zeros((_L,), jnp.float32), cur))
            acc = acc + acc.at[perm8].get(mode="promise_in_bounds")
            acc = acc + acc.at[perm4].get(mode="promise_in_bounds")
            plsc.store_scatter(
                line_v, [store_idx_base + b], acc, mask=store_mask)
            return cur

        lax.fori_loop(0, bpw, bag_body, jnp.int32(-1))

        for cl in range(C):
            bias_c = bias_vec[cl]

            def mean_body(j, _, _bias=bias_c, _cl=cl):
                sums = line_v[pl.ds(_cl * bpw + j * _L, _L)]
                o_lo = off_v[pl.ds(j * _L, _L)]
                o_hi = off_v[pl.ds(j * _L + 1, _L)]
                cnt = (o_hi - o_lo).astype(jnp.float32)
                line_v[pl.ds(_cl * bpw + j * _L, _L)] = (
                    sums / jnp.maximum(cnt, jnp.float32(1.0)) + _bias)
                return 0

            lax.fori_loop(0, bpw // _L, mean_body, 0)
            pltpu.sync_copy(line_v.at[pl.ds(cl * bpw, bpw)],
                            out_hbm.at[pl.ds(cl * B + b0, bpw)])

    return sc_kernel


def kernel(text, offsets, emb_table, fc_w, fc_b):
    T = text.shape[0]
    B = offsets.shape[0]
    V, D = emb_table.shape
    C = fc_w.shape[0]

    wpad = jnp.zeros((D, _CP), jnp.float32).at[:, :C].set(fc_w.T)
    proj4 = _project(emb_table, wpad, 4000)

    offs_ext = jnp.concatenate(
        [offsets.astype(jnp.int32), jnp.full((16,), T, jnp.int32)])
    fcb_pad = jnp.zeros((16,), jnp.float32).at[:C].set(fc_b)

    out_flat = _make_sc_kernel(T, B, V, C)(
        text.astype(jnp.int32), offs_ext, proj4, fcb_pad)
    return out_flat.reshape(C, B).T


# speedup vs baseline: 302.6700x; 1.0347x over previous
import functools

import jax
import jax.numpy as jnp
from jax import lax
from jax.experimental import pallas as pl
from jax.experimental.pallas import tpu as pltpu
from jax.experimental.pallas import tpu_sc as plsc

_NC = 2
_NS = 16
_L = 16
_NW = _NC * _NS

_CP = 4
_CH = 16384


def _proj_body(emb_ref, w_ref, out_ref):
    out_ref[...] = lax.dot_general(
        w_ref[...], emb_ref[...], (((0,), (1,)), ((), ())),
        preferred_element_type=jnp.float32)


def _project(emb, wpad, vb):
    V, D = emb.shape
    return pl.pallas_call(
        _proj_body,
        grid=(pl.cdiv(V, vb),),
        in_specs=[
            pl.BlockSpec((vb, D), lambda i: (i, 0)),
            pl.BlockSpec((D, _CP), lambda i: (0, 0)),
        ],
        out_specs=pl.BlockSpec((_CP, vb), lambda i: (0, i)),
        out_shape=jax.ShapeDtypeStruct((_CP, V), jnp.float32),
    )(emb, wpad)


@functools.lru_cache(maxsize=None)
def _make_sc_kernel(T, B, V, C):
    bpw = B // _NW

    mesh = plsc.VectorSubcoreMesh(
        core_axis_name="c", subcore_axis_name="s",
        num_cores=_NC, num_subcores=_NS)

    @functools.partial(
        pl.kernel,
        out_type=jax.ShapeDtypeStruct((C * B,), jnp.float32),
        mesh=mesh,
        scratch_types=[
            pltpu.VMEM((V,), jnp.float32),
            pltpu.VMEM((_CH,), jnp.int32),
            pltpu.VMEM((bpw + 16,), jnp.int32),
            pltpu.VMEM((bpw,), jnp.float32),
            pltpu.VMEM((16,), jnp.float32),
        ],
        compiler_params=pltpu.CompilerParams(
            needs_layout_passes=False, use_tc_tiling_on_sc=False),
    )
    def sc_kernel(text_hbm, offs_hbm, projT_hbm, fcb_hbm, out_hbm,
                  tab_v, txt_v, off_v, line_v, fcb_v):
        wid = lax.axis_index("s") * _NC + lax.axis_index("c")
        b0 = wid * bpw
        pltpu.sync_copy(offs_hbm.at[pl.ds(b0, bpw + 16)], off_v)
        pltpu.sync_copy(fcb_hbm, fcb_v)
        lane = lax.iota(jnp.int32, _L)
        bias_vec = fcb_v[pl.ds(0, _L)]
        cur_chunk = jnp.int32(-1)

        for c in range(C):
            pltpu.sync_copy(projT_hbm.at[pl.ds(c * V, V)], tab_v)
            bias_c = bias_vec[c]

            def bag_body(b, cur):
                offpair = off_v[pl.ds(b, _L)]
                lo = offpair[0]
                hi = offpair[1]
                i0 = lax.div(lo, 16)
                i1 = lax.div(hi + 15, 16)

                def blk_body(i, carry):
                    acc, cur = carry
                    ck = lax.div(i * 16, _CH)

                    @pl.when(ck != cur)
                    def _():
                        pltpu.sync_copy(
                            text_hbm.at[pl.ds(ck * _CH, _CH)], txt_v)

                    toks = txt_v[pl.ds(i * 16 - ck * _CH, _L)]
                    g = i * 16 + lane
                    m = (g >= lo) & (g < hi)
                    idx = jnp.where(m, toks, 0)
                    vals = plsc.load_gather(tab_v, [idx], mask=m)
                    acc = acc + jnp.where(m, vals, jnp.float32(0.0))
                    return (acc, ck)

                acc, cur = lax.fori_loop(
                    i0, i1, blk_body,
                    (jnp.zeros((_L,), jnp.float32), cur))
                plsc.store_scatter(
                    line_v, [jnp.full((_L,), b, jnp.int32)],
                    jnp.full((_L,), jnp.sum(acc), jnp.float32),
                    mask=lane == 0)
                return cur

            cur_chunk = lax.fori_loop(0, bpw, bag_body, cur_chunk)

            def mean_body(j, _, _bias=bias_c):
                sums = line_v[pl.ds(j * _L, _L)]
                o_lo = off_v[pl.ds(j * _L, _L)]
                o_hi = off_v[pl.ds(j * _L + 1, _L)]
                cnt = (o_hi - o_lo).astype(jnp.float32)
                line_v[pl.ds(j * _L, _L)] = (
                    sums / jnp.maximum(cnt, jnp.float32(1.0)) + _bias)
                return 0

            lax.fori_loop(0, bpw // _L, mean_body, 0)
            pltpu.sync_copy(line_v, out_hbm.at[pl.ds(c * B + b0, bpw)])

    return sc_kernel


def kernel(text, offsets, emb_table, fc_w, fc_b):
    T = text.shape[0]
    B = offsets.shape[0]
    V, D = emb_table.shape
    C = fc_w.shape[0]

    wpad = jnp.zeros((D, _CP), jnp.float32).at[:, :C].set(fc_w.T)
    projT = _project(emb_table, wpad, 12800).reshape(-1)

    offs_ext = jnp.concatenate(
        [offsets.astype(jnp.int32), jnp.full((16,), T, jnp.int32)])
    fcb_pad = jnp.zeros((16,), jnp.float32).at[:C].set(fc_b)

    out_flat = _make_sc_kernel(T, B, V, C)(
        text.astype(jnp.int32), offs_ext, projT, fcb_pad)
    return out_flat.reshape(C, B).T


# speedup vs baseline: 523.1436x; 1.7284x over previous
import functools

import jax
import jax.numpy as jnp
from jax import lax
from jax.experimental import pallas as pl
from jax.experimental.pallas import tpu as pltpu
from jax.experimental.pallas import tpu_sc as plsc

_NC = 2
_NS = 16
_L = 16
_NW = _NC * _NS

_CP = 4
_CH = 16384


def _proj_body(embT_ref, w_ref, out_ref):
    out_ref[...] = lax.dot_general(
        w_ref[...], embT_ref[...], (((0,), (0,)), ((), ())),
        preferred_element_type=jnp.float32)


def _project(embT, wpad, vb):
    D, V = embT.shape
    return pl.pallas_call(
        _proj_body,
        grid=(pl.cdiv(V, vb),),
        in_specs=[
            pl.BlockSpec((D, vb), lambda i: (0, i)),
            pl.BlockSpec((D, _CP), lambda i: (0, 0)),
        ],
        out_specs=pl.BlockSpec((_CP, vb), lambda i: (0, i)),
        out_shape=jax.ShapeDtypeStruct((_CP, V), jnp.float32),
    )(embT, wpad)


@functools.lru_cache(maxsize=None)
def _make_sc_kernel(T, B, V, C):
    bpw = B // _NW

    mesh = plsc.VectorSubcoreMesh(
        core_axis_name="c", subcore_axis_name="s",
        num_cores=_NC, num_subcores=_NS)

    @functools.partial(
        pl.kernel,
        out_type=jax.ShapeDtypeStruct((C * B,), jnp.float32),
        mesh=mesh,
        scratch_types=[
            pltpu.VMEM((V,), jnp.float32),
            pltpu.VMEM((_CH,), jnp.int32),
            pltpu.VMEM((bpw + 16,), jnp.int32),
            pltpu.VMEM((bpw,), jnp.float32),
            pltpu.VMEM((16,), jnp.float32),
        ],
        compiler_params=pltpu.CompilerParams(
            needs_layout_passes=False, use_tc_tiling_on_sc=False),
    )
    def sc_kernel(text_hbm, offs_hbm, projT_hbm, fcb_hbm, out_hbm,
                  tab_v, txt_v, off_v, line_v, fcb_v):
        wid = lax.axis_index("s") * _NC + lax.axis_index("c")
        b0 = wid * bpw
        pltpu.sync_copy(offs_hbm.at[pl.ds(b0, bpw + 16)], off_v)
        pltpu.sync_copy(fcb_hbm, fcb_v)
        lane = lax.iota(jnp.int32, _L)
        bias_vec = fcb_v[pl.ds(0, _L)]
        cur_chunk = jnp.int32(-1)

        for c in range(C):
            pltpu.sync_copy(projT_hbm.at[pl.ds(c * V, V)], tab_v)
            bias_c = bias_vec[c]

            def bag_body(b, cur):
                offpair = off_v[pl.ds(b, _L)]
                lo = offpair[0]
                hi = offpair[1]
                i0 = lax.div(lo, 16)
                i1 = lax.div(hi + 15, 16)

                def blk_body(i, carry):
                    acc, cur = carry
                    ck = lax.div(i * 16, _CH)

                    @pl.when(ck != cur)
                    def _():
                        pltpu.sync_copy(
                            text_hbm.at[pl.ds(ck * _CH, _CH)], txt_v)

                    toks = txt_v[pl.ds(i * 16 - ck * _CH, _L)]
                    g = i * 16 + lane
                    m = (g >= lo) & (g < hi)
                    idx = jnp.where(m, toks, 0)
                    vals = plsc.load_gather(tab_v, [idx], mask=m)
                    acc = acc + jnp.where(m, vals, jnp.float32(0.0))
                    return (acc, ck)

                acc, cur = lax.fori_loop(
                    i0, i1, blk_body,
                    (jnp.zeros((_L,), jnp.float32), cur))
                plsc.store_scatter(
                    line_v, [jnp.full((_L,), b, jnp.int32)],
                    jnp.full((_L,), jnp.sum(acc), jnp.float32),
                    mask=lane == 0)
                return cur

            cur_chunk = lax.fori_loop(0, bpw, bag_body, cur_chunk)

            def mean_body(j, _, _bias=bias_c):
                sums = line_v[pl.ds(j * _L, _L)]
                o_lo = off_v[pl.ds(j * _L, _L)]
                o_hi = off_v[pl.ds(j * _L + 1, _L)]
                cnt = (o_hi - o_lo).astype(jnp.float32)
                line_v[pl.ds(j * _L, _L)] = (
                    sums / jnp.maximum(cnt, jnp.float32(1.0)) + _bias)
                return 0

            lax.fori_loop(0, bpw // _L, mean_body, 0)
            pltpu.sync_copy(line_v, out_hbm.at[pl.ds(c * B + b0, bpw)])

    return sc_kernel


def kernel(text, offsets, emb_table, fc_w, fc_b):
    T = text.shape[0]
    B = offsets.shape[0]
    V, D = emb_table.shape
    C = fc_w.shape[0]

    wpad = jnp.zeros((D, _CP), jnp.float32).at[:, :C].set(fc_w.T)
    projT = _project(emb_table.T, wpad, 12800).reshape(-1)

    offs_ext = jnp.concatenate(
        [offsets.astype(jnp.int32), jnp.full((16,), T, jnp.int32)])
    fcb_pad = jnp.zeros((16,), jnp.float32).at[:C].set(fc_b)

    out_flat = _make_sc_kernel(T, B, V, C)(
        text.astype(jnp.int32), offs_ext, projT, fcb_pad)
    return out_flat.reshape(C, B).T
